# Initial kernel scaffold; baseline (speedup 1.0000x reference)
import jax
import jax.numpy as jnp
from jax.experimental import pallas as pl


def kernel(feat, edge_index, W0, al0, ar0, b0, ln_g, ln_b, W1, al1, ar1, b1):
    raise NotImplementedError("write your pallas kernel here")


# speedup vs baseline: 22.7425x; 22.7425x over previous
import functools

import jax
import jax.numpy as jnp
from jax import lax
from jax.experimental import pallas as pl
from jax.experimental.pallas import tpu as pltpu
from jax.experimental.pallas import tpu_sc as plsc

N = 10000
D = 128
E = 320000
NC = 2
NS = 16
NW = NC * NS
EP = E // NW
CK = 128
NCHUNK = (EP + CK - 1) // CK
EPP = NCHUNK * CK
DEN_SLICE = 640
DEN_PAD = DEN_SLICE * NS
NPAD = DEN_PAD
RPS = NPAD // NS
RCHUNK = 128
M = NPAD
BN = 2048


def _proj_body(h_ref, w_ref, al_ref, ar_ref, ft_ref, el_ref, er_ref):
    i = pl.program_id(0)
    ft = jnp.dot(h_ref[...], w_ref[...], preferred_element_type=jnp.float32)
    ft_ref[...] = ft
    dn = (((1,), (1,)), ((), ()))
    el_ref[:, pl.ds(i * BN, BN)] = lax.dot_general(
        al_ref[...], ft, dn, preferred_element_type=jnp.float32)
    er_ref[:, pl.ds(i * BN, BN)] = lax.dot_general(
        ar_ref[...], ft, dn, preferred_element_type=jnp.float32)


def _project(h, W, al, ar):
    return pl.pallas_call(
        _proj_body,
        grid=(M // BN,),
        in_specs=[pl.BlockSpec((BN, D), lambda i: (i, 0)),
                  pl.BlockSpec((D, D), lambda i: (0, 0)),
                  pl.BlockSpec((1, D), lambda i: (0, 0)),
                  pl.BlockSpec((1, D), lambda i: (0, 0))],
        out_specs=[pl.BlockSpec((BN, D), lambda i: (i, 0)),
                   pl.BlockSpec((1, M), lambda i: (0, 0)),
                   pl.BlockSpec((1, M), lambda i: (0, 0))],
        out_shape=[jax.ShapeDtypeStruct((M, D), jnp.float32),
                   jax.ShapeDtypeStruct((1, M), jnp.float32),
                   jax.ShapeDtypeStruct((1, M), jnp.float32)],
    )(h, W, al, ar)


def _mid_body(p_ref, b_ref, g_ref, bb_ref, w_ref, al_ref, ar_ref,
              ft_ref, el_ref, er_ref):
    x = p_ref[0] + p_ref[1] + b_ref[...]
    mu = jnp.mean(x, axis=-1, keepdims=True)
    var = jnp.mean((x - mu) ** 2, axis=-1, keepdims=True)
    h = (x - mu) / jnp.sqrt(var + 1e-5) * g_ref[...] + bb_ref[...]
    h = jnp.maximum(h, 0.0)
    ft = jnp.dot(h, w_ref[...], preferred_element_type=jnp.float32)
    ft_ref[...] = ft
    i = pl.program_id(0)
    dn = (((1,), (1,)), ((), ()))
    el_ref[:, pl.ds(i * BN, BN)] = lax.dot_general(
        al_ref[...], ft, dn, preferred_element_type=jnp.float32)
    er_ref[:, pl.ds(i * BN, BN)] = lax.dot_general(
        ar_ref[...], ft, dn, preferred_element_type=jnp.float32)


def _mid(p, b, g, bb, W, al, ar):
    return pl.pallas_call(
        _mid_body,
        grid=(M // BN,),
        in_specs=[pl.BlockSpec((NC, BN, D), lambda i: (0, i, 0)),
                  pl.BlockSpec((1, D), lambda i: (0, 0)),
                  pl.BlockSpec((1, D), lambda i: (0, 0)),
                  pl.BlockSpec((1, D), lambda i: (0, 0)),
                  pl.BlockSpec((D, D), lambda i: (0, 0)),
                  pl.BlockSpec((1, D), lambda i: (0, 0)),
                  pl.BlockSpec((1, D), lambda i: (0, 0))],
        out_specs=[pl.BlockSpec((BN, D), lambda i: (i, 0)),
                   pl.BlockSpec((1, M), lambda i: (0, 0)),
                   pl.BlockSpec((1, M), lambda i: (0, 0))],
        out_shape=[jax.ShapeDtypeStruct((M, D), jnp.float32),
                   jax.ShapeDtypeStruct((1, M), jnp.float32),
                   jax.ShapeDtypeStruct((1, M), jnp.float32)],
    )(p, b, g, bb, W, al, ar)


def _fin_body(p_ref, b_ref, o_ref):
    o_ref[...] = p_ref[0] + p_ref[1] + b_ref[...]


def _final(p, b):
    return pl.pallas_call(
        _fin_body,
        grid=(M // BN,),
        in_specs=[pl.BlockSpec((NC, BN, D), lambda i: (0, i, 0)),
                  pl.BlockSpec((1, D), lambda i: (0, 0))],
        out_specs=pl.BlockSpec((BN, D), lambda i: (i, 0)),
        out_shape=jax.ShapeDtypeStruct((M, D), jnp.float32),
    )(p, b)


def _edge_softmax(el, er, srcT, dstT):
    mesh = plsc.VectorSubcoreMesh(core_axis_name="c", subcore_axis_name="s", num_cores=NC, num_subcores=NS)

    @functools.partial(
        pl.kernel, mesh=mesh,
        compiler_params=pltpu.CompilerParams(needs_layout_passes=False),
        out_type=[jax.ShapeDtypeStruct((NW, NCHUNK, CK), jnp.float32),
                  jax.ShapeDtypeStruct((NC, N), jnp.float32)],
        scratch_types=[pltpu.VMEM((N,), jnp.float32),
                       pltpu.VMEM((N,), jnp.float32),
                       pltpu.VMEM((NCHUNK, CK), jnp.int32),
                       pltpu.VMEM((NCHUNK, CK), jnp.int32),
                       pltpu.VMEM((NCHUNK, CK), jnp.float32),
                       pltpu.VMEM((DEN_SLICE,), jnp.float32),
                       pltpu.VMEM_SHARED((DEN_PAD,), jnp.float32)])
    def k(el_hbm, er_hbm, src_hbm, dst_hbm, ee_hbm, den_hbm,
          el_v, er_v, src_v, dst_v, ee_v, zero_v, den_sh):
        cid = lax.axis_index("c")
        sid = lax.axis_index("s")
        wid = cid * NS + sid
        pltpu.sync_copy(el_hbm, el_v)
        pltpu.sync_copy(er_hbm, er_v)
        pltpu.sync_copy(src_hbm.at[wid], src_v)
        pltpu.sync_copy(dst_hbm.at[wid], dst_v)

        def zbody(i, _):
            zero_v[pl.ds(i * 16, 16)] = jnp.zeros((16,), jnp.float32)
            return 0
        lax.fori_loop(0, DEN_SLICE // 16, zbody, 0)
        pltpu.sync_copy(zero_v, den_sh.at[pl.ds(sid * DEN_SLICE, DEN_SLICE)])
        plsc.subcore_barrier()

        def cbody(c, _):
            base_c = c * CK
            for kk in range(CK // 16):
                sl = pl.ds(kk * 16, 16)
                s16 = src_v[c, sl]
                d16 = dst_v[c, sl]
                ev = plsc.load_gather(el_v, [s16]) + plsc.load_gather(er_v, [d16])
                ev = jnp.where(ev >= 0, ev, ev * 0.2)
                ee = jnp.exp(ev)
                pos = base_c + kk * 16 + lax.iota(jnp.int32, 16)
                ee_v[c, sl] = jnp.where(pos < EP, ee, 0.0)
            return 0
        lax.fori_loop(0, NCHUNK, cbody, 0)

        def sbody(c, _):
            pltpu.sync_copy(ee_v.at[c], den_sh.at[dst_v.at[c]], add=True)
            return 0
        lax.fori_loop(0, NCHUNK, sbody, 0)
        plsc.subcore_barrier()

        pltpu.sync_copy(ee_v, ee_hbm.at[wid])

        @pl.when(sid == 0)
        def _():
            pltpu.sync_copy(den_sh.at[pl.ds(0, N)], el_v)
            pltpu.sync_copy(el_v, den_hbm.at[cid])

    return k(el, er, srcT, dstT)


def _alpha(ee, den, dstT):
    mesh = plsc.VectorSubcoreMesh(core_axis_name="c", subcore_axis_name="s", num_cores=NC, num_subcores=NS)

    @functools.partial(
        pl.kernel, mesh=mesh,
        compiler_params=pltpu.CompilerParams(needs_layout_passes=False),
        out_type=jax.ShapeDtypeStruct((NW, NCHUNK, CK), jnp.float32),
        scratch_types=[pltpu.VMEM((N,), jnp.float32),
                       pltpu.VMEM((N,), jnp.float32),
                       pltpu.VMEM((NCHUNK, CK), jnp.int32),
                       pltpu.VMEM((NCHUNK, CK), jnp.float32)])
    def k(ee_hbm, den_hbm, dst_hbm, alpha_hbm, den_v, den2_v, dst_v, av):
        cid = lax.axis_index("c")
        sid = lax.axis_index("s")
        wid = cid * NS + sid
        pltpu.sync_copy(den_hbm.at[0], den_v)
        pltpu.sync_copy(den_hbm.at[1], den2_v)
        pltpu.sync_copy(dst_hbm.at[wid], dst_v)
        pltpu.sync_copy(ee_hbm.at[wid], av)

        def dbody(i, _):
            sl = pl.ds(i * 16, 16)
            den_v[sl] = den_v[sl] + den2_v[sl]
            return 0
        lax.fori_loop(0, N // 16, dbody, 0)

        def abody(c, _):
            for kk in range(CK // 16):
                sl = pl.ds(kk * 16, 16)
                dg = plsc.load_gather(den_v, [dst_v[c, sl]])
                av[c, sl] = av[c, sl] / (dg + 1e-9)
            return 0
        lax.fori_loop(0, NCHUNK, abody, 0)
        pltpu.sync_copy(av, alpha_hbm.at[wid])

    return k(ee, den, dstT)


def _aggregate(ft, alpha, srcT, dstT):
    mesh = plsc.VectorSubcoreMesh(core_axis_name="c", subcore_axis_name="s", num_cores=NC, num_subcores=NS)

    @functools.partial(
        pl.kernel, mesh=mesh,
        compiler_params=pltpu.CompilerParams(needs_layout_passes=False),
        out_type=jax.ShapeDtypeStruct((NC, NPAD, D), jnp.float32),
        scratch_types=[pltpu.VMEM((NCHUNK, CK), jnp.int32),
                       pltpu.VMEM((NCHUNK, CK), jnp.int32),
                       pltpu.VMEM((NCHUNK, CK), jnp.float32),
                       pltpu.VMEM((CK, D), jnp.float32),
                       pltpu.VMEM_SHARED((NPAD, D), jnp.float32),
                       pltpu.SemaphoreType.DMA])
    def k(ft_hbm, alpha_hbm, src_hbm, dst_hbm, out_hbm,
          src_v, dst_v, alpha_v, rows_v, rst_sh, sem):
        cid = lax.axis_index("c")
        sid = lax.axis_index("s")
        wid = cid * NS + sid
        pltpu.sync_copy(src_hbm.at[wid], src_v)
        pltpu.sync_copy(dst_hbm.at[wid], dst_v)
        pltpu.sync_copy(alpha_hbm.at[wid], alpha_v)

        def zbody(r, _):
            for kk in range(D // 16):
                rows_v[r, pl.ds(kk * 16, 16)] = jnp.zeros((16,), jnp.float32)
            return 0
        lax.fori_loop(0, CK, zbody, 0)
        row_lo = sid * RPS
        for j in range(RPS // RCHUNK):
            pltpu.sync_copy(rows_v.at[pl.ds(0, RCHUNK)],
                            rst_sh.at[pl.ds(row_lo + j * RCHUNK, RCHUNK)])
        plsc.subcore_barrier()

        def mbody(c, _):
            pltpu.async_copy(ft_hbm.at[src_v.at[c]], rows_v, sem).wait()

            def rbody(m, _):
                a16 = alpha_v[c, pl.ds(m * 16, 16)]
                for rr in range(16):
                    r = m * 16 + rr
                    av = jnp.full((16,), a16[rr], jnp.float32)
                    for kk in range(D // 16):
                        sl = pl.ds(kk * 16, 16)
                        rows_v[r, sl] = rows_v[r, sl] * av
                return 0
            lax.fori_loop(0, CK // 16, rbody, 0)
            pltpu.sync_copy(rows_v, rst_sh.at[dst_v.at[c]], add=True)
            return 0
        lax.fori_loop(0, NCHUNK, mbody, 0)
        plsc.subcore_barrier()

        for j in range(RPS // RCHUNK):
            lo = row_lo + j * RCHUNK
            pltpu.sync_copy(rst_sh.at[pl.ds(lo, RCHUNK)],
                            rows_v.at[pl.ds(0, RCHUNK)])
            pltpu.sync_copy(rows_v.at[pl.ds(0, RCHUNK)],
                            out_hbm.at[cid, pl.ds(lo, RCHUNK)])

    return k(ft, alpha, srcT, dstT)


def kernel(feat, edge_index, W0, al0, ar0, b0, ln_g, ln_b, W1, al1, ar1, b1):
    src = edge_index[0].reshape(NW, EP)
    dst = edge_index[1].reshape(NW, EP)
    pad = ((0, 0), (0, EPP - EP))
    srcT = jnp.pad(src, pad).reshape(NW, NCHUNK, CK)
    dstT = jnp.pad(dst, pad).reshape(NW, NCHUNK, CK)
    b0r = b0.reshape(1, D)
    b1r = b1.reshape(1, D)
    gr = ln_g.reshape(1, D)
    br = ln_b.reshape(1, D)

    featP = jnp.pad(feat, ((0, M - N), (0, 0)))
    ft0, el0, er0 = _project(featP, W0, al0, ar0)
    ee0, den0 = _edge_softmax(el0[0, :N], er0[0, :N], srcT, dstT)
    a0 = _alpha(ee0, den0, dstT)
    p0 = _aggregate(ft0, a0, srcT, dstT)
    ft1, el1, er1 = _mid(p0, b0r, gr, br, W1, al1, ar1)
    ee1, den1 = _edge_softmax(el1[0, :N], er1[0, :N], srcT, dstT)
    a1 = _alpha(ee1, den1, dstT)
    p1 = _aggregate(ft1, a1, srcT, dstT)
    return _final(p1, b1r)[:N]
